# Initial kernel scaffold; baseline (speedup 1.0000x reference)
import jax
import jax.numpy as jnp
from jax.experimental import pallas as pl


def kernel(x, Wq, bq, key_embed, value_embed, W1, b1, W2, b2, access_counts, success_counts):
    raise NotImplementedError("write your pallas kernel here")


# speedup vs baseline: 6.2262x; 6.2262x over previous
import dataclasses
import functools

import jax
import jax.numpy as jnp
import numpy as np
from jax import lax
from jax.experimental import pallas as pl
from jax.experimental.pallas import tpu as pltpu
from jax.experimental.pallas import tpu_sc as plsc

D = 1024
NK = 16384
K = 32
ER = 0.1
SCALE = 1.0 / np.sqrt(D)

TB = 128
KB = 2048
NKB = NK // KB

NTOK = 4096
NT = NTOK // TB

SC_NC = 2
SC_NS = 16
SC_L = 16
NW = SC_NC * SC_NS
TPW = NTOK // NW


def _score_topk_body(x_ref, wqt_ref, bq_ref, kt_ref, ac_ref, su_ref,
                     w_out_ref, i_out_ref, q_scr, s_scr):
    kb = pl.program_id(1)

    @pl.when(kb == 0)
    def _():
        q_scr[...] = (
            jnp.dot(x_ref[...], wqt_ref[...],
                    preferred_element_type=jnp.float32) + bq_ref[...])

    ac = ac_ref[...]
    su = su_ref[...]
    bias = ER * ((1.0 - ER) * (su / (ac + 1e-10)) + ER / (ac + 1.0))
    s_scr[kb] = (
        jnp.dot(q_scr[...], kt_ref[...],
                preferred_element_type=jnp.float32) * SCALE + bias)

    @pl.when(kb == NKB - 1)
    def _():
        col3 = (lax.broadcasted_iota(jnp.int32, (NKB, TB, KB), 0) * KB
                + lax.broadcasted_iota(jnp.int32, (NKB, TB, KB), 2))
        ii = lax.broadcasted_iota(jnp.int32, (TB, K), 1)

        def body(i, carry):
            tv, ti = carry
            s = s_scr[...]
            m = jnp.max(jnp.max(s, axis=2), axis=0)
            mb = m[None, :, None]
            cand = jnp.where(s == mb, col3, jnp.int32(2 ** 30))
            mcol = jnp.min(jnp.min(cand, axis=2), axis=0)
            s_scr[...] = jnp.where(col3 == mcol[None, :, None],
                                   -jnp.inf, s)
            tv = jnp.where(ii == i, m[:, None], tv)
            ti = jnp.where(ii == i, mcol[:, None], ti)
            return tv, ti

        tv0 = jnp.full((TB, K), -jnp.inf, jnp.float32)
        ti0 = jnp.zeros((TB, K), jnp.int32)
        tv, ti = lax.fori_loop(0, K, body, (tv0, ti0))
        e = jnp.exp(tv - tv[:, 0:1])
        w_out_ref[...] = e / jnp.sum(e, axis=1, keepdims=True)
        i_out_ref[...] = ti


def _stage_a(xf, wqt, bq2, kt, ac2, su2, interpret=False):
    return pl.pallas_call(
        _score_topk_body,
        grid=(NT, NKB),
        in_specs=[
            pl.BlockSpec((TB, D), lambda t, k: (t, 0)),
            pl.BlockSpec((D, D), lambda t, k: (0, 0)),
            pl.BlockSpec((1, D), lambda t, k: (0, 0)),
            pl.BlockSpec((D, KB), lambda t, k: (0, k)),
            pl.BlockSpec((1, KB), lambda t, k: (0, k)),
            pl.BlockSpec((1, KB), lambda t, k: (0, k)),
        ],
        out_specs=[
            pl.BlockSpec((TB, K), lambda t, k: (t, 0)),
            pl.BlockSpec((TB, K), lambda t, k: (t, 0)),
        ],
        out_shape=[
            jax.ShapeDtypeStruct((NTOK, K), jnp.float32),
            jax.ShapeDtypeStruct((NTOK, K), jnp.int32),
        ],
        scratch_shapes=[
            pltpu.VMEM((TB, D), jnp.float32),
            pltpu.VMEM((NKB, TB, KB), jnp.float32),
        ],
        compiler_params=pltpu.CompilerParams(
            dimension_semantics=("parallel", "arbitrary")),
        interpret=interpret,
    )(xf, wqt, bq2, kt, ac2, su2)


def _gather_combine_body(v_hbm, i_hbm, w_hbm, o_hbm,
                         idx_v, w_v, wb_v, rows_v, orow_v, gsem):
    wid = lax.axis_index("s") * SC_NC + lax.axis_index("c")
    base = wid * TPW
    pltpu.sync_copy(i_hbm.at[pl.ds(base * K, TPW * K)], idx_v)
    pltpu.sync_copy(w_hbm.at[pl.ds(base * K, TPW * K)], w_v)

    @pl.loop(0, TPW)
    def _(t):
        tk = t * K
        pltpu.async_copy(v_hbm.at[idx_v.at[pl.ds(tk, K)]], rows_v,
                         gsem).wait()

        @pl.loop(0, K)
        def _(k):
            wb_v[k] = plsc.load_gather(
                w_v, [tk + k + jnp.zeros((SC_L,), jnp.int32)])

        @pl.loop(0, D, step=SC_L)
        def _(c):
            acc = jnp.zeros((SC_L,), jnp.float32)
            for k in range(K):
                acc = acc + wb_v[k] * rows_v[k, pl.ds(c, SC_L)]
            orow_v[pl.ds(c, SC_L)] = acc

        pltpu.sync_copy(orow_v, o_hbm.at[base + t])


def _stage_b(value_embed, idx_flat, w_flat):
    mesh = plsc.VectorSubcoreMesh(core_axis_name="c", subcore_axis_name="s")
    cp = pltpu.CompilerParams()
    if "needs_layout_passes" in pltpu.CompilerParams.__dataclass_fields__:
        cp = dataclasses.replace(cp, needs_layout_passes=False)
    knl = pl.kernel(
        _gather_combine_body,
        out_type=jax.ShapeDtypeStruct((NTOK, D), jnp.float32),
        mesh=mesh,
        scratch_types=[
            pltpu.VMEM((TPW * K,), jnp.int32),
            pltpu.VMEM((TPW * K,), jnp.float32),
            pltpu.VMEM((K, SC_L), jnp.float32),
            pltpu.VMEM((K, D), jnp.float32),
            pltpu.VMEM((D,), jnp.float32),
            pltpu.SemaphoreType.DMA,
        ],
        compiler_params=cp,
    )
    return knl(value_embed, idx_flat, w_flat)


CB = 512


def _conf_body(o_ref, w1t_ref, b1_ref, w2t_ref, b2_ref, c_ref):
    h = jnp.maximum(
        jnp.dot(o_ref[...], w1t_ref[...],
                preferred_element_type=jnp.float32) + b1_ref[...], 0.0)
    z = (jnp.dot(h, w2t_ref[...], preferred_element_type=jnp.float32)
         + b2_ref[...])
    c_ref[...] = 1.0 / (1.0 + jnp.exp(-z))


def _stage_c(out_flat, w1t, b12, w2t, b22, interpret=False):
    return pl.pallas_call(
        _conf_body,
        grid=(NTOK // CB,),
        in_specs=[
            pl.BlockSpec((CB, D), lambda t: (t, 0)),
            pl.BlockSpec((D, D // 2), lambda t: (0, 0)),
            pl.BlockSpec((1, D // 2), lambda t: (0, 0)),
            pl.BlockSpec((D // 2, 1), lambda t: (0, 0)),
            pl.BlockSpec((1, 1), lambda t: (0, 0)),
        ],
        out_specs=pl.BlockSpec((CB, 1), lambda t: (t, 0)),
        out_shape=jax.ShapeDtypeStruct((NTOK, 1), jnp.float32),
        compiler_params=pltpu.CompilerParams(
            dimension_semantics=("parallel",)),
        interpret=interpret,
    )(out_flat, w1t, b12, w2t, b22)


def kernel(x, Wq, bq, key_embed, value_embed, W1, b1, W2, b2,
           access_counts, success_counts):
    B, S, _ = x.shape
    xf = x.reshape(B * S, D)
    weights, indices = _stage_a(
        xf, Wq.T, bq.reshape(1, D), key_embed.T,
        access_counts.reshape(1, NK), success_counts.reshape(1, NK))
    out_flat = _stage_b(value_embed, indices.reshape(-1),
                        weights.reshape(-1))
    conf = _stage_c(out_flat, W1.T, b1.reshape(1, D // 2), W2.T,
                    b2.reshape(1, 1))
    return (out_flat.reshape(B, S, D), conf.reshape(B, S, 1),
            indices.reshape(B, S, K))
